# baseline (speedup 1.0000x reference)
import jax
import jax.numpy as jnp
from jax.experimental import pallas as pl
from jax.experimental.pallas import tpu as pltpu
from jax.experimental.pallas import tpu_sc as plsc

HEADS = 8
DH = 64
KD, KH, KW = 8, 16, 16
D, H, W = 16, 32, 32
NPOS = D * H * W
NQ = 8 * 16 * 16
NSEL = KD * KH * KW
INNER = HEADS * DH


def _ctx_body(ctx_ref, g_ref, b_ref, wkv_ref, kv_ref, kd_ref, kh_ref, kw_ref):
    d = pl.program_id(0)
    x = ctx_ref[...]
    mean = jnp.mean(x, axis=0, keepdims=True)
    var = jnp.mean((x - mean) ** 2, axis=0, keepdims=True)
    xn = g_ref[...] * (x - mean) / (jnp.sqrt(var) + 1e-6) + b_ref[...]
    kv = jax.lax.dot_general(xn, wkv_ref[...], (((0,), (1,)), ((), ())),
                             preferred_element_type=jnp.float32, precision=jax.lax.Precision.HIGHEST)
    kd_parts, kh_parts, kw_parts = [], [], []
    for h in range(HEADS):
        k_h = kv[:, h * DH:(h + 1) * DH]
        n = jnp.sqrt(jnp.sum(k_h * k_h, axis=1, keepdims=True))
        k_hn = k_h / jnp.maximum(n, 1e-12)
        kv_ref[h, :, :DH] = k_hn
        kv_ref[h, :, DH:] = kv[:, INNER + h * DH:INNER + (h + 1) * DH]
        ka = jnp.abs(k_hn)
        kd_parts.append(jnp.sum(ka, axis=0, keepdims=True))
        ka3 = ka.reshape(H, W, DH)
        kh_parts.append(jnp.sum(ka3, axis=1))
        kw_parts.append(jnp.sum(ka3, axis=0))
    kd_ref[0] = jnp.concatenate(kd_parts, axis=1)
    kh_row = jnp.concatenate(kh_parts, axis=1)
    kw_row = jnp.concatenate(kw_parts, axis=1)

    @pl.when(d == 0)
    def _():
        kh_ref[...] = kh_row
        kw_ref[...] = kw_row

    @pl.when(d != 0)
    def _():
        kh_ref[...] += kh_row
        kw_ref[...] += kw_row


def _q_body(qs_ref, g_ref, b_ref, wq_ref, q_ref, qp_ref):
    x = qs_ref[...]
    mean = jnp.mean(x, axis=0, keepdims=True)
    var = jnp.mean((x - mean) ** 2, axis=0, keepdims=True)
    xn = g_ref[...] * (x - mean) / (jnp.sqrt(var) + 1e-6) + b_ref[...]
    q = jax.lax.dot_general(xn, wq_ref[...], (((0,), (1,)), ((), ())),
                            preferred_element_type=jnp.float32, precision=jax.lax.Precision.HIGHEST)
    qp_parts = []
    for h in range(HEADS):
        q_h = q[:, h * DH:(h + 1) * DH]
        n = jnp.sqrt(jnp.sum(q_h * q_h, axis=1, keepdims=True))
        q_hn = q_h / jnp.maximum(n, 1e-12)
        q_ref[h] = q_hn
        qp_parts.append(jnp.sum(q_hn, axis=0, keepdims=True))
    qp_ref[...] = jnp.concatenate(qp_parts, axis=1)


def _scores(sums, qp):
    prod = sums * qp
    cols = [jnp.sum(prod[:, h * DH:(h + 1) * DH], axis=1, keepdims=True)
            for h in range(HEADS)]
    return jnp.concatenate(cols, axis=1)


def _topk_into(s, k, out_ref):
    L = s.shape[0]
    iota = jax.lax.broadcasted_iota(jnp.int32, s.shape, 0)
    for t in range(k):
        m = jnp.max(s, axis=0, keepdims=True)
        am = jnp.min(jnp.where(s == m, iota, L), axis=0, keepdims=True)
        out_ref[pl.ds(t, 1), :] = am
        s = jnp.where(iota == am, -jnp.inf, s)


def _topk_body(qp_ref, kd_ref, kh_ref, kw_ref, id_ref, ih_ref, iw_ref):
    qp = qp_ref[...]
    _topk_into(_scores(kd_ref[...], qp), KD, id_ref)
    _topk_into(_scores(kh_ref[...], qp), KH, ih_ref)
    _topk_into(_scores(kw_ref[...], qp), KW, iw_ref)


def _attn_body(q_ref, kv_ref, o_ref):
    q = q_ref[0]
    k = kv_ref[0, :, :DH]
    v = kv_ref[0, :, DH:]
    sim = jax.lax.dot_general(q, k, (((1,), (1,)), ((), ())),
                              preferred_element_type=jnp.float32, precision=jax.lax.Precision.HIGHEST)
    m = jnp.max(sim, axis=1, keepdims=True)
    e = jnp.exp(sim - m)
    s = jnp.sum(e, axis=1, keepdims=True)
    o = jnp.dot(e, v, preferred_element_type=jnp.float32, precision=jax.lax.Precision.HIGHEST)
    o_ref[0] = o / s


def _out_body(attn_ref, wout_ref, g_ref, b_ref, qs_ref, gamma_ref, o_ref):
    attn = jnp.concatenate([attn_ref[h] for h in range(HEADS)], axis=1)
    o = jax.lax.dot_general(wout_ref[...], attn,
                            (((1,), (1,)), ((), ())),
                            preferred_element_type=jnp.float32, precision=jax.lax.Precision.HIGHEST)
    mean = jnp.mean(o, axis=0, keepdims=True)
    var = jnp.mean((o - mean) ** 2, axis=0, keepdims=True)
    on = g_ref[...] * (o - mean) / (jnp.sqrt(var) + 1e-6) + b_ref[...]
    o_ref[...] = gamma_ref[0, 0] * on + qs_ref[...]


def _sc_gather(table, indices):
    n_idx = indices.shape[1]
    win = 128
    mesh = plsc.VectorSubcoreMesh(core_axis_name="core",
                                  subcore_axis_name="subcore")

    @pl.kernel(out_type=jax.ShapeDtypeStruct((n_idx, table.shape[1]),
                                             table.dtype),
               mesh=mesh)
    def kern(x_hbm, i_hbm, o_hbm):
        def body(i_vmem, o_vmem):
            pltpu.sync_copy(x_hbm.at[i_vmem.at[0]], o_vmem)

        pltpu.emit_pipeline(
            body,
            grid=(n_idx // win,),
            in_specs=[pl.BlockSpec((1, win), index_map=lambda i: (0, i))],
            out_specs=[pl.BlockSpec((win, table.shape[1]),
                                    index_map=lambda i: (i, 0))],
            core_axis_name="subcore",
            dimension_semantics=(pltpu.PARALLEL,),
        )(i_hbm, o_hbm)

    return kern(table, indices)


def kernel(query_source, context, cn_gamma, cn_beta, qs_gamma, qs_beta,
           W_kv, W_q, W_out, on_gamma, on_beta, gamma):
    dim = context.shape[1]
    ctx = context.reshape(dim, NPOS)
    qs_cm = query_source.reshape(dim, NQ)
    cn_g = cn_gamma.reshape(dim, 1)
    cn_b = cn_beta.reshape(dim, 1)
    qs_g = qs_gamma.reshape(dim, 1)
    qs_b = qs_beta.reshape(dim, 1)
    on_g = on_gamma.reshape(dim, 1)
    on_b = on_beta.reshape(dim, 1)

    kv_rows, ksum_d, ksum_h, ksum_w = pl.pallas_call(
        _ctx_body,
        grid=(D,),
        in_specs=[
            pl.BlockSpec((dim, H * W), lambda d: (0, d)),
            pl.BlockSpec((dim, 1), lambda d: (0, 0)),
            pl.BlockSpec((dim, 1), lambda d: (0, 0)),
            pl.BlockSpec((2 * INNER, dim), lambda d: (0, 0)),
        ],
        out_specs=[
            pl.BlockSpec((HEADS, H * W, 2 * DH), lambda d: (0, d, 0)),
            pl.BlockSpec((1, 1, INNER), lambda d: (d, 0, 0)),
            pl.BlockSpec((H, INNER), lambda d: (0, 0)),
            pl.BlockSpec((W, INNER), lambda d: (0, 0)),
        ],
        out_shape=[
            jax.ShapeDtypeStruct((HEADS, NPOS, 2 * DH), jnp.float32),
            jax.ShapeDtypeStruct((D, 1, INNER), jnp.float32),
            jax.ShapeDtypeStruct((H, INNER), jnp.float32),
            jax.ShapeDtypeStruct((W, INNER), jnp.float32),
        ],
    )(ctx, cn_g, cn_b, W_kv)
    ksum_d = ksum_d.reshape(D, INNER)

    q_rows, q_probe = pl.pallas_call(
        _q_body,
        out_shape=[
            jax.ShapeDtypeStruct((HEADS, NQ, DH), jnp.float32),
            jax.ShapeDtypeStruct((1, INNER), jnp.float32),
        ],
    )(qs_cm, qs_g, qs_b, W_q)

    id_t, ih_t, iw_t = pl.pallas_call(
        _topk_body,
        out_shape=[
            jax.ShapeDtypeStruct((KD, HEADS), jnp.int32),
            jax.ShapeDtypeStruct((KH, HEADS), jnp.int32),
            jax.ShapeDtypeStruct((KW, HEADS), jnp.int32),
        ],
    )(q_probe, ksum_d, ksum_h, ksum_w)

    idx_d = id_t.T
    idx_h = ih_t.T
    idx_w = iw_t.T
    hoff = (jnp.arange(HEADS, dtype=jnp.int32) * NPOS)[:, None, None, None]
    flat = (hoff + idx_d[:, :, None, None] * (H * W)
            + idx_h[:, None, :, None] * W + idx_w[:, None, None, :])
    flat = flat.reshape(1, HEADS * NSEL)

    kv_sel = _sc_gather(kv_rows.reshape(HEADS * NPOS, 2 * DH), flat)
    kv_sel = kv_sel.reshape(HEADS, NSEL, 2 * DH)

    attn_out = pl.pallas_call(
        _attn_body,
        grid=(HEADS,),
        in_specs=[
            pl.BlockSpec((1, NQ, DH), lambda h: (h, 0, 0)),
            pl.BlockSpec((1, NSEL, 2 * DH), lambda h: (h, 0, 0)),
        ],
        out_specs=pl.BlockSpec((1, NQ, DH), lambda h: (h, 0, 0)),
        out_shape=jax.ShapeDtypeStruct((HEADS, NQ, DH), jnp.float32),
    )(q_rows, kv_sel)

    out_cm = pl.pallas_call(
        _out_body,
        out_shape=jax.ShapeDtypeStruct((dim, NQ), jnp.float32),
    )(attn_out, W_out, on_g, on_b, qs_cm, gamma.reshape(1, 1))

    return out_cm.reshape(query_source.shape)


# speedup vs baseline: 3.0608x; 3.0608x over previous
import jax
import jax.numpy as jnp
from jax.experimental import pallas as pl
from jax.experimental.pallas import tpu as pltpu
from jax.experimental.pallas import tpu_sc as plsc

HEADS = 8
DH = 64
KD, KH, KW = 8, 16, 16
D, H, W = 16, 32, 32
NPOS = D * H * W
NQ = 8 * 16 * 16
NSEL = KD * KH * KW
INNER = HEADS * DH


def _ctx_body(ctx_ref, g_ref, b_ref, wkv_ref, kv_ref, kd_ref, kh_ref, kw_ref):
    d = pl.program_id(0)
    x = ctx_ref[...]
    mean = jnp.mean(x, axis=0, keepdims=True)
    var = jnp.mean((x - mean) ** 2, axis=0, keepdims=True)
    xn = g_ref[...] * (x - mean) / (jnp.sqrt(var) + 1e-6) + b_ref[...]
    kv = jax.lax.dot_general(xn, wkv_ref[...], (((0,), (1,)), ((), ())),
                             preferred_element_type=jnp.float32)
    kd_parts, kh_parts, kw_parts = [], [], []
    for h in range(HEADS):
        k_h = kv[:, h * DH:(h + 1) * DH]
        n = jnp.sqrt(jnp.sum(k_h * k_h, axis=1, keepdims=True))
        k_hn = k_h / jnp.maximum(n, 1e-12)
        kv_ref[h, :, :DH] = k_hn
        kv_ref[h, :, DH:] = kv[:, INNER + h * DH:INNER + (h + 1) * DH]
        ka = jnp.abs(k_hn)
        kd_parts.append(jnp.sum(ka, axis=0, keepdims=True))
        ka3 = ka.reshape(H, W, DH)
        kh_parts.append(jnp.sum(ka3, axis=1))
        kw_parts.append(jnp.sum(ka3, axis=0))
    kd_ref[0] = jnp.concatenate(kd_parts, axis=1)
    kh_row = jnp.concatenate(kh_parts, axis=1)
    kw_row = jnp.concatenate(kw_parts, axis=1)

    @pl.when(d == 0)
    def _():
        kh_ref[...] = kh_row
        kw_ref[...] = kw_row

    @pl.when(d != 0)
    def _():
        kh_ref[...] += kh_row
        kw_ref[...] += kw_row


def _q_body(qs_ref, g_ref, b_ref, wq_ref, q_ref, qp_ref):
    x = qs_ref[...]
    mean = jnp.mean(x, axis=0, keepdims=True)
    var = jnp.mean((x - mean) ** 2, axis=0, keepdims=True)
    xn = g_ref[...] * (x - mean) / (jnp.sqrt(var) + 1e-6) + b_ref[...]
    q = jax.lax.dot_general(xn, wq_ref[...], (((0,), (1,)), ((), ())),
                            preferred_element_type=jnp.float32)
    qp_parts = []
    for h in range(HEADS):
        q_h = q[:, h * DH:(h + 1) * DH]
        n = jnp.sqrt(jnp.sum(q_h * q_h, axis=1, keepdims=True))
        q_hn = q_h / jnp.maximum(n, 1e-12)
        q_ref[h] = q_hn
        qp_parts.append(jnp.sum(q_hn, axis=0, keepdims=True))
    qp_ref[...] = jnp.concatenate(qp_parts, axis=1)


def _scores(sums, qp):
    sums = sums.astype(jnp.bfloat16).astype(jnp.float32)
    qp = qp.astype(jnp.bfloat16).astype(jnp.float32)
    prod = sums * qp
    cols = [jnp.sum(prod[:, h * DH:(h + 1) * DH], axis=1, keepdims=True)
            for h in range(HEADS)]
    return jnp.concatenate(cols, axis=1)


def _topk_into(s, k, out_ref):
    L = s.shape[0]
    iota = jax.lax.broadcasted_iota(jnp.int32, s.shape, 0)
    for t in range(k):
        m = jnp.max(s, axis=0, keepdims=True)
        am = jnp.min(jnp.where(s == m, iota, L), axis=0, keepdims=True)
        out_ref[pl.ds(t, 1), :] = am
        s = jnp.where(iota == am, -jnp.inf, s)


def _topk_body(qp_ref, kd_ref, kh_ref, kw_ref, id_ref, ih_ref, iw_ref):
    qp = qp_ref[...]
    _topk_into(_scores(kd_ref[...], qp), KD, id_ref)
    _topk_into(_scores(kh_ref[...], qp), KH, ih_ref)
    _topk_into(_scores(kw_ref[...], qp), KW, iw_ref)


def _attn_body(q_ref, kv_ref, o_ref):
    q = q_ref[0]
    k = kv_ref[0, :, :DH]
    v = kv_ref[0, :, DH:]
    sim = jax.lax.dot_general(q, k, (((1,), (1,)), ((), ())),
                              preferred_element_type=jnp.float32)
    m = jnp.max(sim, axis=1, keepdims=True)
    e = jnp.exp(sim - m)
    s = jnp.sum(e, axis=1, keepdims=True)
    o = jnp.dot(e, v, preferred_element_type=jnp.float32)
    o_ref[0] = o / s


def _out_body(attn_ref, wout_ref, g_ref, b_ref, qs_ref, gamma_ref, o_ref):
    attn = jnp.concatenate([attn_ref[h] for h in range(HEADS)], axis=1)
    o = jax.lax.dot_general(wout_ref[...], attn,
                            (((1,), (1,)), ((), ())),
                            preferred_element_type=jnp.float32)
    mean = jnp.mean(o, axis=0, keepdims=True)
    var = jnp.mean((o - mean) ** 2, axis=0, keepdims=True)
    on = g_ref[...] * (o - mean) / (jnp.sqrt(var) + 1e-6) + b_ref[...]
    o_ref[...] = gamma_ref[0, 0] * on + qs_ref[...]


def _sc_gather(table, indices):
    n_idx = indices.shape[1]
    win = 128
    mesh = plsc.VectorSubcoreMesh(core_axis_name="core",
                                  subcore_axis_name="subcore")

    @pl.kernel(out_type=jax.ShapeDtypeStruct((n_idx, table.shape[1]),
                                             table.dtype),
               mesh=mesh)
    def kern(x_hbm, i_hbm, o_hbm):
        def body(i_vmem, o_vmem):
            pltpu.sync_copy(x_hbm.at[i_vmem.at[0]], o_vmem)

        pltpu.emit_pipeline(
            body,
            grid=(n_idx // win,),
            in_specs=[pl.BlockSpec((1, win), index_map=lambda i: (0, i))],
            out_specs=[pl.BlockSpec((win, table.shape[1]),
                                    index_map=lambda i: (i, 0))],
            core_axis_name="subcore",
            dimension_semantics=(pltpu.PARALLEL,),
        )(i_hbm, o_hbm)

    return kern(table, indices)


def kernel(query_source, context, cn_gamma, cn_beta, qs_gamma, qs_beta,
           W_kv, W_q, W_out, on_gamma, on_beta, gamma):
    dim = context.shape[1]
    ctx = context.reshape(dim, NPOS)
    qs_cm = query_source.reshape(dim, NQ)
    cn_g = cn_gamma.reshape(dim, 1)
    cn_b = cn_beta.reshape(dim, 1)
    qs_g = qs_gamma.reshape(dim, 1)
    qs_b = qs_beta.reshape(dim, 1)
    on_g = on_gamma.reshape(dim, 1)
    on_b = on_beta.reshape(dim, 1)

    kv_rows, ksum_d, ksum_h, ksum_w = pl.pallas_call(
        _ctx_body,
        grid=(D,),
        in_specs=[
            pl.BlockSpec((dim, H * W), lambda d: (0, d)),
            pl.BlockSpec((dim, 1), lambda d: (0, 0)),
            pl.BlockSpec((dim, 1), lambda d: (0, 0)),
            pl.BlockSpec((2 * INNER, dim), lambda d: (0, 0)),
        ],
        out_specs=[
            pl.BlockSpec((HEADS, H * W, 2 * DH), lambda d: (0, d, 0)),
            pl.BlockSpec((1, 1, INNER), lambda d: (d, 0, 0)),
            pl.BlockSpec((H, INNER), lambda d: (0, 0)),
            pl.BlockSpec((W, INNER), lambda d: (0, 0)),
        ],
        out_shape=[
            jax.ShapeDtypeStruct((HEADS, NPOS, 2 * DH), jnp.float32),
            jax.ShapeDtypeStruct((D, 1, INNER), jnp.float32),
            jax.ShapeDtypeStruct((H, INNER), jnp.float32),
            jax.ShapeDtypeStruct((W, INNER), jnp.float32),
        ],
    )(ctx, cn_g, cn_b, W_kv)
    ksum_d = ksum_d.reshape(D, INNER)

    q_rows, q_probe = pl.pallas_call(
        _q_body,
        out_shape=[
            jax.ShapeDtypeStruct((HEADS, NQ, DH), jnp.float32),
            jax.ShapeDtypeStruct((1, INNER), jnp.float32),
        ],
    )(qs_cm, qs_g, qs_b, W_q)

    id_t, ih_t, iw_t = pl.pallas_call(
        _topk_body,
        out_shape=[
            jax.ShapeDtypeStruct((KD, HEADS), jnp.int32),
            jax.ShapeDtypeStruct((KH, HEADS), jnp.int32),
            jax.ShapeDtypeStruct((KW, HEADS), jnp.int32),
        ],
    )(q_probe, ksum_d, ksum_h, ksum_w)

    idx_d = id_t.T
    idx_h = ih_t.T
    idx_w = iw_t.T
    hoff = (jnp.arange(HEADS, dtype=jnp.int32) * NPOS)[:, None, None, None]
    flat = (hoff + idx_d[:, :, None, None] * (H * W)
            + idx_h[:, None, :, None] * W + idx_w[:, None, None, :])
    flat = flat.reshape(1, HEADS * NSEL)

    kv_sel = _sc_gather(kv_rows.reshape(HEADS * NPOS, 2 * DH), flat)
    kv_sel = kv_sel.reshape(HEADS, NSEL, 2 * DH)

    attn_out = pl.pallas_call(
        _attn_body,
        grid=(HEADS,),
        in_specs=[
            pl.BlockSpec((1, NQ, DH), lambda h: (h, 0, 0)),
            pl.BlockSpec((1, NSEL, 2 * DH), lambda h: (h, 0, 0)),
        ],
        out_specs=pl.BlockSpec((1, NQ, DH), lambda h: (h, 0, 0)),
        out_shape=jax.ShapeDtypeStruct((HEADS, NQ, DH), jnp.float32),
    )(q_rows, kv_sel)

    out_cm = pl.pallas_call(
        _out_body,
        out_shape=jax.ShapeDtypeStruct((dim, NQ), jnp.float32),
    )(attn_out, W_out, on_g, on_b, qs_cm, gamma.reshape(1, 1))

    return out_cm.reshape(query_source.shape)


# speedup vs baseline: 3.0725x; 1.0038x over previous
import jax
import jax.numpy as jnp
from jax.experimental import pallas as pl
from jax.experimental.pallas import tpu as pltpu
from jax.experimental.pallas import tpu_sc as plsc

HEADS = 8
DH = 64
KD, KH, KW = 8, 16, 16
D, H, W = 16, 32, 32
NPOS = D * H * W
NQ = 8 * 16 * 16
NSEL = KD * KH * KW
INNER = HEADS * DH


def _ctx_body(ctx_ref, g_ref, b_ref, wkv_ref, kv_ref, kd_ref, kh_ref, kw_ref):
    d = pl.program_id(0)
    x = ctx_ref[...]
    mean = jnp.mean(x, axis=0, keepdims=True)
    var = jnp.mean((x - mean) ** 2, axis=0, keepdims=True)
    xn = g_ref[...] * (x - mean) / (jnp.sqrt(var) + 1e-6) + b_ref[...]
    kv = jax.lax.dot_general(xn, wkv_ref[...], (((0,), (1,)), ((), ())),
                             preferred_element_type=jnp.float32)
    kd_parts, kh_parts, kw_parts = [], [], []
    for h in range(HEADS):
        k_h = kv[:, h * DH:(h + 1) * DH]
        n = jnp.sqrt(jnp.sum(k_h * k_h, axis=1, keepdims=True))
        k_hn = k_h / jnp.maximum(n, 1e-12)
        kv_ref[h, :, :DH] = k_hn
        kv_ref[h, :, DH:] = kv[:, INNER + h * DH:INNER + (h + 1) * DH]
        ka = jnp.abs(k_hn)
        kd_parts.append(jnp.sum(ka, axis=0, keepdims=True))
        ka3 = ka.reshape(H, W, DH)
        kh_parts.append(jnp.sum(ka3, axis=1))
        kw_parts.append(jnp.sum(ka3, axis=0))
    kd_ref[0] = jnp.concatenate(kd_parts, axis=1)
    kh_row = jnp.concatenate(kh_parts, axis=1)
    kw_row = jnp.concatenate(kw_parts, axis=1)

    @pl.when(d == 0)
    def _():
        kh_ref[...] = kh_row
        kw_ref[...] = kw_row

    @pl.when(d != 0)
    def _():
        kh_ref[...] += kh_row
        kw_ref[...] += kw_row


def _q_body(qs_ref, g_ref, b_ref, wq_ref, q_ref, qp_ref):
    x = qs_ref[...]
    mean = jnp.mean(x, axis=0, keepdims=True)
    var = jnp.mean((x - mean) ** 2, axis=0, keepdims=True)
    xn = g_ref[...] * (x - mean) / (jnp.sqrt(var) + 1e-6) + b_ref[...]
    q = jax.lax.dot_general(xn, wq_ref[...], (((0,), (1,)), ((), ())),
                            preferred_element_type=jnp.float32)
    qp_parts = []
    for h in range(HEADS):
        q_h = q[:, h * DH:(h + 1) * DH]
        n = jnp.sqrt(jnp.sum(q_h * q_h, axis=1, keepdims=True))
        q_hn = q_h / jnp.maximum(n, 1e-12)
        q_ref[h] = q_hn.astype(jnp.bfloat16)
        qp_parts.append(jnp.sum(q_hn, axis=0, keepdims=True))
    qp_ref[...] = jnp.concatenate(qp_parts, axis=1)


def _scores(sums, qp):
    sums = sums.astype(jnp.bfloat16).astype(jnp.float32)
    qp = qp.astype(jnp.bfloat16).astype(jnp.float32)
    prod = sums * qp
    cols = [jnp.sum(prod[:, h * DH:(h + 1) * DH], axis=1, keepdims=True)
            for h in range(HEADS)]
    return jnp.concatenate(cols, axis=1)


def _topk_into(s, k, out_ref):
    L = s.shape[0]
    iota = jax.lax.broadcasted_iota(jnp.int32, s.shape, 0)
    for t in range(k):
        m = jnp.max(s, axis=0, keepdims=True)
        am = jnp.min(jnp.where(s == m, iota, L), axis=0, keepdims=True)
        out_ref[pl.ds(t, 1), :] = am
        s = jnp.where(iota == am, -jnp.inf, s)


def _topk_body(qp_ref, kd_ref, kh_ref, kw_ref, id_ref, ih_ref, iw_ref):
    qp = qp_ref[...]
    _topk_into(_scores(kd_ref[...], qp), KD, id_ref)
    _topk_into(_scores(kh_ref[...], qp), KH, ih_ref)
    _topk_into(_scores(kw_ref[...], qp), KW, iw_ref)


def _attn_body(q_ref, kv_ref, o_ref):
    q = q_ref[0]
    k = kv_ref[0, :, :DH]
    v = kv_ref[0, :, DH:]
    sim = jax.lax.dot_general(q, k, (((1,), (1,)), ((), ())),
                              preferred_element_type=jnp.float32)
    m = jnp.max(sim, axis=1, keepdims=True)
    e = jnp.exp(sim - m)
    s = jnp.sum(e, axis=1, keepdims=True)
    o = jnp.dot(e, v, preferred_element_type=jnp.float32)
    o_ref[0] = o / s


def _out_body(attn_ref, wout_ref, g_ref, b_ref, qs_ref, gamma_ref, o_ref):
    attn = jnp.concatenate([attn_ref[h] for h in range(HEADS)], axis=1)
    o = jax.lax.dot_general(wout_ref[...], attn,
                            (((1,), (1,)), ((), ())),
                            preferred_element_type=jnp.float32)
    mean = jnp.mean(o, axis=0, keepdims=True)
    var = jnp.mean((o - mean) ** 2, axis=0, keepdims=True)
    on = g_ref[...] * (o - mean) / (jnp.sqrt(var) + 1e-6) + b_ref[...]
    o_ref[...] = gamma_ref[0, 0] * on + qs_ref[...]


def _sc_gather(table, indices):
    n_idx = indices.shape[1]
    win = 128
    mesh = plsc.VectorSubcoreMesh(core_axis_name="core",
                                  subcore_axis_name="subcore")

    @pl.kernel(out_type=jax.ShapeDtypeStruct((n_idx, table.shape[1]),
                                             table.dtype),
               mesh=mesh)
    def kern(x_hbm, i_hbm, o_hbm):
        def body(i_vmem, o_vmem):
            pltpu.sync_copy(x_hbm.at[i_vmem.at[0]], o_vmem)

        pltpu.emit_pipeline(
            body,
            grid=(n_idx // win,),
            in_specs=[pl.BlockSpec((1, win), index_map=lambda i: (0, i))],
            out_specs=[pl.BlockSpec((win, table.shape[1]),
                                    index_map=lambda i: (i, 0))],
            core_axis_name="subcore",
            dimension_semantics=(pltpu.PARALLEL,),
        )(i_hbm, o_hbm)

    return kern(table, indices)


def kernel(query_source, context, cn_gamma, cn_beta, qs_gamma, qs_beta,
           W_kv, W_q, W_out, on_gamma, on_beta, gamma):
    dim = context.shape[1]
    ctx = context.reshape(dim, NPOS)
    qs_cm = query_source.reshape(dim, NQ)
    cn_g = cn_gamma.reshape(dim, 1)
    cn_b = cn_beta.reshape(dim, 1)
    qs_g = qs_gamma.reshape(dim, 1)
    qs_b = qs_beta.reshape(dim, 1)
    on_g = on_gamma.reshape(dim, 1)
    on_b = on_beta.reshape(dim, 1)

    kv_rows, ksum_d, ksum_h, ksum_w = pl.pallas_call(
        _ctx_body,
        grid=(D,),
        in_specs=[
            pl.BlockSpec((dim, H * W), lambda d: (0, d)),
            pl.BlockSpec((dim, 1), lambda d: (0, 0)),
            pl.BlockSpec((dim, 1), lambda d: (0, 0)),
            pl.BlockSpec((2 * INNER, dim), lambda d: (0, 0)),
        ],
        out_specs=[
            pl.BlockSpec((HEADS, H * W, 2 * DH), lambda d: (0, d, 0)),
            pl.BlockSpec((1, 1, INNER), lambda d: (d, 0, 0)),
            pl.BlockSpec((H, INNER), lambda d: (0, 0)),
            pl.BlockSpec((W, INNER), lambda d: (0, 0)),
        ],
        out_shape=[
            jax.ShapeDtypeStruct((HEADS, NPOS, 2 * DH), jnp.float32),
            jax.ShapeDtypeStruct((D, 1, INNER), jnp.float32),
            jax.ShapeDtypeStruct((H, INNER), jnp.float32),
            jax.ShapeDtypeStruct((W, INNER), jnp.float32),
        ],
    )(ctx, cn_g, cn_b, W_kv)
    ksum_d = ksum_d.reshape(D, INNER)

    q_rows, q_probe = pl.pallas_call(
        _q_body,
        out_shape=[
            jax.ShapeDtypeStruct((HEADS, NQ, DH), jnp.bfloat16),
            jax.ShapeDtypeStruct((1, INNER), jnp.float32),
        ],
    )(qs_cm, qs_g, qs_b, W_q)

    id_t, ih_t, iw_t = pl.pallas_call(
        _topk_body,
        out_shape=[
            jax.ShapeDtypeStruct((KD, HEADS), jnp.int32),
            jax.ShapeDtypeStruct((KH, HEADS), jnp.int32),
            jax.ShapeDtypeStruct((KW, HEADS), jnp.int32),
        ],
    )(q_probe, ksum_d, ksum_h, ksum_w)

    idx_d = id_t.T
    idx_h = ih_t.T
    idx_w = iw_t.T
    hoff = (jnp.arange(HEADS, dtype=jnp.int32) * NPOS)[:, None, None, None]
    flat = (hoff + idx_d[:, :, None, None] * (H * W)
            + idx_h[:, None, :, None] * W + idx_w[:, None, None, :])
    flat = flat.reshape(1, HEADS * NSEL)

    kv_sel = _sc_gather(kv_rows.reshape(HEADS * NPOS, 2 * DH), flat)
    kv_sel = kv_sel.reshape(HEADS, NSEL, 2 * DH)

    attn_out = pl.pallas_call(
        _attn_body,
        grid=(HEADS,),
        in_specs=[
            pl.BlockSpec((1, NQ, DH), lambda h: (h, 0, 0)),
            pl.BlockSpec((1, NSEL, 2 * DH), lambda h: (h, 0, 0)),
        ],
        out_specs=pl.BlockSpec((1, NQ, DH), lambda h: (h, 0, 0)),
        out_shape=jax.ShapeDtypeStruct((HEADS, NQ, DH), jnp.float32),
    )(q_rows, kv_sel)

    out_cm = pl.pallas_call(
        _out_body,
        out_shape=jax.ShapeDtypeStruct((dim, NQ), jnp.float32),
    )(attn_out, W_out, on_g, on_b, qs_cm, gamma.reshape(1, 1))

    return out_cm.reshape(query_source.shape)


# speedup vs baseline: 3.5494x; 1.1552x over previous
import jax
import jax.numpy as jnp
from jax.experimental import pallas as pl
from jax.experimental.pallas import tpu as pltpu
from jax.experimental.pallas import tpu_sc as plsc

HEADS = 8
DH = 64
KD, KH, KW = 8, 16, 16
D, H, W = 16, 32, 32
NPOS = D * H * W
NQ = 8 * 16 * 16
NSEL = KD * KH * KW
INNER = HEADS * DH
HIGH = jax.lax.Precision.HIGHEST


def _head_eye(dtype=jnp.float32):
    i = jax.lax.broadcasted_iota(jnp.int32, (INNER, HEADS), 0)
    j = jax.lax.broadcasted_iota(jnp.int32, (INNER, HEADS), 1)
    return (i // DH == j).astype(dtype)


def _chan_norm_rows(x, g, b):
    mean = jnp.mean(x, axis=1, keepdims=True)
    var = jnp.mean((x - mean) ** 2, axis=1, keepdims=True)
    return g * (x - mean) / (jnp.sqrt(var) + 1e-6) + b


def _l2norm_heads(t):
    n2 = jax.lax.dot_general(t * t, _head_eye(), (((1,), (0,)), ((), ())),
                             preferred_element_type=jnp.float32,
                             precision=HIGH)
    n = jnp.maximum(jnp.sqrt(n2), 1e-12)
    parts = [t[:, h * DH:(h + 1) * DH] / n[:, h:h + 1] for h in range(HEADS)]
    return jnp.concatenate(parts, axis=1)


def _ctx_body(ctx_ref, g_ref, b_ref, wk_ref, cn_ref, kd_ref, kh_ref, kw_ref):
    d = pl.program_id(0)
    x = ctx_ref[...]
    xn = _chan_norm_rows(x, g_ref[...], b_ref[...])
    cn_ref[...] = xn
    k = jax.lax.dot_general(xn, wk_ref[...], (((1,), (1,)), ((), ())),
                            preferred_element_type=jnp.float32)
    ka = jnp.abs(_l2norm_heads(k))
    kd_ref[0] = jnp.sum(ka, axis=0, keepdims=True)
    ka3 = ka.reshape(H, W, INNER)
    kh_row = jnp.sum(ka3, axis=1)
    kw_row = jnp.sum(ka3, axis=0)

    @pl.when(d == 0)
    def _():
        kh_ref[...] = kh_row
        kw_ref[...] = kw_row

    @pl.when(d != 0)
    def _():
        kh_ref[...] += kh_row
        kw_ref[...] += kw_row


def _q_body(qs_ref, g_ref, b_ref, wq_ref, q_ref, qp_ref):
    x = qs_ref[...]
    xn = _chan_norm_rows(x, g_ref[...], b_ref[...])
    q = jax.lax.dot_general(xn, wq_ref[...], (((1,), (1,)), ((), ())),
                            preferred_element_type=jnp.float32)
    qn = _l2norm_heads(q)
    for h in range(HEADS):
        q_ref[h] = qn[:, h * DH:(h + 1) * DH].astype(jnp.bfloat16)
    qp_ref[...] = jnp.sum(qn, axis=0, keepdims=True)


def _scores(sums, qp):
    sums = sums.astype(jnp.bfloat16).astype(jnp.float32)
    qp = qp.astype(jnp.bfloat16).astype(jnp.float32)
    prod = sums * qp
    cols = [jnp.sum(prod[:, h * DH:(h + 1) * DH], axis=1, keepdims=True)
            for h in range(HEADS)]
    return jnp.concatenate(cols, axis=1)


def _topk_into(s, k, out_ref):
    L = s.shape[0]
    iota = jax.lax.broadcasted_iota(jnp.int32, s.shape, 0)
    for t in range(k):
        m = jnp.max(s, axis=0, keepdims=True)
        am = jnp.min(jnp.where(s == m, iota, L), axis=0, keepdims=True)
        out_ref[pl.ds(t, 1), :] = am
        s = jnp.where(iota == am, -jnp.inf, s)


def _topk_body(qp_ref, kd_ref, kh_ref, kw_ref, id_ref, ih_ref, iw_ref):
    qp = qp_ref[...]
    _topk_into(_scores(kd_ref[...], qp), KD, id_ref)
    _topk_into(_scores(kh_ref[...], qp), KH, ih_ref)
    _topk_into(_scores(kw_ref[...], qp), KW, iw_ref)


def _attn_body(q_ref, cs_ref, wk_ref, wv_ref, o_ref):
    cs = cs_ref[0]
    k = jax.lax.dot_general(cs, wk_ref[...], (((1,), (1,)), ((), ())),
                            preferred_element_type=jnp.float32)
    n2 = jax.lax.dot_general(k * k, jnp.ones((DH, 8), jnp.float32),
                             (((1,), (0,)), ((), ())),
                             preferred_element_type=jnp.float32,
                             precision=HIGH)[:, :1]
    kn = (k / jnp.maximum(jnp.sqrt(n2), 1e-12)).astype(jnp.bfloat16)
    v = jax.lax.dot_general(cs, wv_ref[...], (((1,), (1,)), ((), ())),
                            preferred_element_type=jnp.float32)
    q = q_ref[0]
    sim = jax.lax.dot_general(q, kn, (((1,), (1,)), ((), ())),
                              preferred_element_type=jnp.float32)
    m = jnp.max(sim, axis=1, keepdims=True)
    e = jnp.exp(sim - m)
    s = jnp.sum(e, axis=1, keepdims=True)
    o = jnp.dot(e, v, preferred_element_type=jnp.float32)
    o_ref[0] = o / s


def _out_body(attn_ref, wout_ref, g_ref, b_ref, qs_ref, gamma_ref, o_ref):
    attn = jnp.concatenate([attn_ref[h] for h in range(HEADS)], axis=1)
    o = jax.lax.dot_general(wout_ref[...], attn,
                            (((1,), (1,)), ((), ())),
                            preferred_element_type=jnp.float32)
    mean = jnp.mean(o, axis=0, keepdims=True)
    var = jnp.mean((o - mean) ** 2, axis=0, keepdims=True)
    on = g_ref[...] * (o - mean) / (jnp.sqrt(var) + 1e-6) + b_ref[...]
    o_ref[...] = gamma_ref[0, 0] * on + qs_ref[...]


def _sc_gather(table, indices):
    n_idx = indices.shape[1]
    win = 128
    mesh = plsc.VectorSubcoreMesh(core_axis_name="core",
                                  subcore_axis_name="subcore")

    @pl.kernel(out_type=jax.ShapeDtypeStruct((n_idx, table.shape[1]),
                                             table.dtype),
               mesh=mesh)
    def kern(x_hbm, i_hbm, o_hbm):
        def body(i_vmem, o_vmem):
            pltpu.sync_copy(x_hbm.at[i_vmem.at[0]], o_vmem)

        pltpu.emit_pipeline(
            body,
            grid=(n_idx // win,),
            in_specs=[pl.BlockSpec((1, win), index_map=lambda i: (0, i))],
            out_specs=[pl.BlockSpec((win, table.shape[1]),
                                    index_map=lambda i: (i, 0))],
            core_axis_name="subcore",
            dimension_semantics=(pltpu.PARALLEL,),
        )(i_hbm, o_hbm)

    return kern(table, indices)


def kernel(query_source, context, cn_gamma, cn_beta, qs_gamma, qs_beta,
           W_kv, W_q, W_out, on_gamma, on_beta, gamma):
    dim = context.shape[1]
    ctx_t = context.reshape(dim, NPOS).T
    qs_t = query_source.reshape(dim, NQ).T
    qs_cm = query_source.reshape(dim, NQ)
    cn_g = cn_gamma.reshape(1, dim)
    cn_b = cn_beta.reshape(1, dim)
    qs_g = qs_gamma.reshape(1, dim)
    qs_b = qs_beta.reshape(1, dim)
    on_g = on_gamma.reshape(dim, 1)
    on_b = on_beta.reshape(dim, 1)
    W_k = W_kv[:INNER]
    W_v = W_kv[INNER:]

    ctxn_rows, ksum_d, ksum_h, ksum_w = pl.pallas_call(
        _ctx_body,
        grid=(D,),
        in_specs=[
            pl.BlockSpec((H * W, dim), lambda d: (d, 0)),
            pl.BlockSpec((1, dim), lambda d: (0, 0)),
            pl.BlockSpec((1, dim), lambda d: (0, 0)),
            pl.BlockSpec((INNER, dim), lambda d: (0, 0)),
        ],
        out_specs=[
            pl.BlockSpec((H * W, dim), lambda d: (d, 0)),
            pl.BlockSpec((1, 1, INNER), lambda d: (d, 0, 0)),
            pl.BlockSpec((H, INNER), lambda d: (0, 0)),
            pl.BlockSpec((W, INNER), lambda d: (0, 0)),
        ],
        out_shape=[
            jax.ShapeDtypeStruct((NPOS, dim), jnp.float32),
            jax.ShapeDtypeStruct((D, 1, INNER), jnp.float32),
            jax.ShapeDtypeStruct((H, INNER), jnp.float32),
            jax.ShapeDtypeStruct((W, INNER), jnp.float32),
        ],
    )(ctx_t, cn_g, cn_b, W_k)
    ksum_d = ksum_d.reshape(D, INNER)

    q_rows, q_probe = pl.pallas_call(
        _q_body,
        out_shape=[
            jax.ShapeDtypeStruct((HEADS, NQ, DH), jnp.bfloat16),
            jax.ShapeDtypeStruct((1, INNER), jnp.float32),
        ],
    )(qs_t, qs_g, qs_b, W_q)

    id_t, ih_t, iw_t = pl.pallas_call(
        _topk_body,
        out_shape=[
            jax.ShapeDtypeStruct((KD, HEADS), jnp.int32),
            jax.ShapeDtypeStruct((KH, HEADS), jnp.int32),
            jax.ShapeDtypeStruct((KW, HEADS), jnp.int32),
        ],
    )(q_probe, ksum_d, ksum_h, ksum_w)

    idx_d = id_t.T
    idx_h = ih_t.T
    idx_w = iw_t.T
    flat = (idx_d[:, :, None, None] * (H * W)
            + idx_h[:, None, :, None] * W + idx_w[:, None, None, :])
    flat = flat.reshape(1, HEADS * NSEL)

    ctx_sel = _sc_gather(ctxn_rows, flat)
    ctx_sel = ctx_sel.reshape(HEADS, NSEL, dim)

    attn_out = pl.pallas_call(
        _attn_body,
        grid=(HEADS,),
        in_specs=[
            pl.BlockSpec((1, NQ, DH), lambda h: (h, 0, 0)),
            pl.BlockSpec((1, NSEL, dim), lambda h: (h, 0, 0)),
            pl.BlockSpec((DH, dim), lambda h: (h, 0)),
            pl.BlockSpec((DH, dim), lambda h: (h, 0)),
        ],
        out_specs=pl.BlockSpec((1, NQ, DH), lambda h: (h, 0, 0)),
        out_shape=jax.ShapeDtypeStruct((HEADS, NQ, DH), jnp.float32),
    )(q_rows, ctx_sel, W_k, W_v)

    out_cm = pl.pallas_call(
        _out_body,
        out_shape=jax.ShapeDtypeStruct((dim, NQ), jnp.float32),
    )(attn_out, W_out, on_g, on_b, qs_cm, gamma.reshape(1, 1))

    return out_cm.reshape(query_source.shape)
